# Initial kernel scaffold; baseline (speedup 1.0000x reference)
import jax
import jax.numpy as jnp
from jax.experimental import pallas as pl


def kernel(x_sub, Mu, edge_index_sub, subset_idx, W, lin_l_w, lin_l_b, lin_r_w, lin_r_b, att):
    raise NotImplementedError("write your pallas kernel here")


# speedup vs baseline: 9.6770x; 9.6770x over previous
import functools

import jax
import jax.numpy as jnp
from jax import lax
from jax.experimental import pallas as pl
from jax.experimental.pallas import tpu as pltpu
from jax.experimental.pallas import tpu_sc as plsc

N = 10000
E = 320000
C = 32
G = 128
H = 8
NC, NS = 2, 16
NW = NC * NS

WROWS_PER_W = 8
WACT = 12
WPAD = WACT * WROWS_PER_W * 128
EROWS_PER_W = 80
EPAD = NW * EROWS_PER_W * 128
EROWS = EPAD // 128
CHUNK_ROWS = 8
NCHUNK = EROWS_PER_W // CHUNK_ROWS
NPAD = 10240
RPW = NPAD // NS

_mesh = plsc.VectorSubcoreMesh(core_axis_name="c", subcore_axis_name="s")


@functools.partial(
    pl.kernel,
    out_type=jax.ShapeDtypeStruct((WPAD, C), jnp.float32),
    mesh=_mesh,
    scratch_types=[
        pltpu.VMEM((WROWS_PER_W, 128), jnp.int32),
        pltpu.VMEM((WROWS_PER_W * 128, C), jnp.float32),
        pltpu.SemaphoreType.DMA,
    ],
    compiler_params=pltpu.CompilerParams(use_tc_tiling_on_sc=False, needs_layout_passes=False),
)
def _wgather(idx_hbm, w_hbm, out_hbm, idx_v, rows_v, sem):
    wid = lax.axis_index("s") * NC + lax.axis_index("c")

    @pl.when(wid < WACT)
    def _():
        pltpu.sync_copy(
            idx_hbm.at[pl.ds(wid * WROWS_PER_W, WROWS_PER_W), :], idx_v)
        cps = [
            pltpu.async_copy(w_hbm.at[idx_v.at[j]],
                             rows_v.at[pl.ds(j * 128, 128)], sem)
            for j in range(WROWS_PER_W)
        ]
        for cp in cps:
            cp.wait()
        pltpu.sync_copy(rows_v, out_hbm.at[pl.ds(wid * WROWS_PER_W * 128,
                                                 WROWS_PER_W * 128)])


def _dense_body(x_ref, mu_ref, ws_ref, llw_ref, llb_ref, lrw_ref, lrb_ref,
                p_ref, l_ref, xl_ref, xr_ref, cos_ref, ll_ref):
    i = pl.program_id(0)
    x = x_ref[...]
    xl_ref[...] = jnp.dot(x, llw_ref[...],
                          preferred_element_type=jnp.float32) + llb_ref[...]
    xr_ref[...] = jnp.dot(x, lrw_ref[...],
                          preferred_element_type=jnp.float32) + lrb_ref[...]
    w = ws_ref[...]
    ew = jnp.exp(w - jnp.max(w, axis=1, keepdims=True))
    p = ew / jnp.sum(ew, axis=1, keepdims=True)
    p_ref[...] = p
    l_ref[...] = jnp.log(p + 1e-8)
    xn = x / jnp.maximum(
        jnp.sqrt(jnp.sum(x * x, axis=1, keepdims=True)), 1e-12)
    mu = mu_ref[...]
    mun = mu / jnp.maximum(
        jnp.sqrt(jnp.sum(mu * mu, axis=1, keepdims=True)), 1e-12)
    cos = lax.dot_general(xn, mun, (((1,), (1,)), ((), ())),
                          preferred_element_type=jnp.float32)
    cos_ref[...] = cos

    @pl.when(i == 0)
    def _():
        ll_ref[0, 0] = 0.0

    ll_ref[0, 0] += jnp.sum(p * cos) * 10.0


_dense_call = pl.pallas_call(
    _dense_body,
    grid=(10,),
    in_specs=[
        pl.BlockSpec((1000, G), lambda i: (i, 0)),
        pl.BlockSpec((C, G), lambda i: (0, 0)),
        pl.BlockSpec((1000, C), lambda i: (i, 0)),
        pl.BlockSpec((G, H), lambda i: (0, 0)),
        pl.BlockSpec((1, H), lambda i: (0, 0)),
        pl.BlockSpec((G, H), lambda i: (0, 0)),
        pl.BlockSpec((1, H), lambda i: (0, 0)),
    ],
    out_specs=[
        pl.BlockSpec((1000, C), lambda i: (i, 0)),
        pl.BlockSpec((1000, C), lambda i: (i, 0)),
        pl.BlockSpec((1000, H), lambda i: (i, 0)),
        pl.BlockSpec((1000, H), lambda i: (i, 0)),
        pl.BlockSpec((1000, C), lambda i: (i, 0)),
        pl.BlockSpec((1, 1), lambda i: (0, 0), memory_space=pltpu.SMEM),
    ],
    out_shape=[
        jax.ShapeDtypeStruct((N, C), jnp.float32),
        jax.ShapeDtypeStruct((N, C), jnp.float32),
        jax.ShapeDtypeStruct((N, H), jnp.float32),
        jax.ShapeDtypeStruct((N, H), jnp.float32),
        jax.ShapeDtypeStruct((N, C), jnp.float32),
        jax.ShapeDtypeStruct((1, 1), jnp.float32),
    ],
)


@functools.partial(
    pl.kernel,
    out_type=jax.ShapeDtypeStruct((NC, NPAD, C), jnp.float32),
    mesh=_mesh,
    scratch_types=[
        pltpu.VMEM((CHUNK_ROWS, 128), jnp.int32),
        pltpu.VMEM((CHUNK_ROWS, 128), jnp.int32),
        pltpu.VMEM((CHUNK_ROWS * 128, H), jnp.float32),
        pltpu.VMEM((CHUNK_ROWS * 128, H), jnp.float32),
        pltpu.VMEM((CHUNK_ROWS * 128, C), jnp.float32),
        pltpu.VMEM((H,), jnp.float32),
        pltpu.VMEM_SHARED((NPAD, C), jnp.float32),
        pltpu.SemaphoreType.DMA,
    ],
    compiler_params=pltpu.CompilerParams(use_tc_tiling_on_sc=False, needs_layout_passes=False),
)
def _edges(eidx_hbm, xl_hbm, xr_hbm, p_hbm, att_hbm, vout_hbm,
           sidx, didx, xlr, xrr, prows, attv, vsh, sem):
    cid = lax.axis_index("c")
    sid = lax.axis_index("s")
    wid = sid * NC + cid
    pltpu.sync_copy(att_hbm, attv)
    atts = [plsc.load_gather(attv, [jnp.full((16,), k, jnp.int32)])
            for k in range(H)]

    def zbody(r, carry):
        z = jnp.zeros((16,), jnp.float32)
        prows[r, pl.ds(0, 16)] = z
        prows[r, pl.ds(16, 16)] = z
        return carry

    lax.fori_loop(0, RPW, zbody, 0)
    pltpu.sync_copy(prows.at[pl.ds(0, RPW)], vsh.at[pl.ds(sid * RPW, RPW)])
    plsc.subcore_barrier()

    row0 = wid * EROWS_PER_W

    def chunk_body(c, carry):
        r0 = row0 + c * CHUNK_ROWS
        pltpu.sync_copy(eidx_hbm.at[0, pl.ds(r0, CHUNK_ROWS), :], sidx)
        pltpu.sync_copy(eidx_hbm.at[1, pl.ds(r0, CHUNK_ROWS), :], didx)
        cps = []
        for j in range(CHUNK_ROWS):
            cps.append(pltpu.async_copy(
                xl_hbm.at[sidx.at[j]], xlr.at[pl.ds(j * 128, 128)], sem))
            cps.append(pltpu.async_copy(
                xr_hbm.at[didx.at[j]], xrr.at[pl.ds(j * 128, 128)], sem))
            cps.append(pltpu.async_copy(
                p_hbm.at[sidx.at[j]], prows.at[pl.ds(j * 128, 128)], sem))
        for cp in cps:
            cp.wait()
        gbase = r0 * 128

        def gbody(g, carry2):
            ids = g * 16 + lax.iota(jnp.int32, 16)
            gid = gbase + ids
            mask = gid < E
            acc = jnp.zeros((16,), jnp.float32)
            for k in range(H):
                kf = jnp.full((16,), k, jnp.int32)
                a = plsc.load_gather(xlr, [ids, kf])
                b = plsc.load_gather(xrr, [ids, kf])
                s = a + b
                acc = acc + atts[k] * jnp.maximum(s, 0.2 * s)
            ex = jnp.where(mask, jnp.exp(acc), 0.0)
            for k in range(C):
                kf = jnp.full((16,), k, jnp.int32)
                pk = plsc.load_gather(prows, [ids, kf])
                plsc.store_scatter(prows, [ids, kf], ex * pk)
            return carry2

        lax.fori_loop(0, CHUNK_ROWS * 8, gbody, 0)
        for j in range(CHUNK_ROWS):
            pltpu.sync_copy(prows.at[pl.ds(j * 128, 128)],
                            vsh.at[didx.at[j]], add=True)
        return carry

    lax.fori_loop(0, NCHUNK, chunk_body, 0)
    plsc.subcore_barrier()
    pltpu.sync_copy(vsh.at[pl.ds(sid * RPW, RPW)],
                    vout_hbm.at[cid, pl.ds(sid * RPW, RPW), :])


def _final_body(l_ref, v_ref, ce_ref):
    i = pl.program_id(0)
    v = v_ref[0] + v_ref[1]
    den = jnp.sum(v, axis=1, keepdims=True) + 1e-16
    part = jnp.sum(l_ref[...] * v / den)

    @pl.when(i == 0)
    def _():
        ce_ref[0, 0] = 0.0

    ce_ref[0, 0] += part


_final_call = pl.pallas_call(
    _final_body,
    grid=(10,),
    in_specs=[
        pl.BlockSpec((1000, C), lambda i: (i, 0)),
        pl.BlockSpec((NC, 1000, C), lambda i: (0, i, 0)),
    ],
    out_specs=pl.BlockSpec((1, 1), lambda i: (0, 0), memory_space=pltpu.SMEM),
    out_shape=jax.ShapeDtypeStruct((1, 1), jnp.float32),
)


def kernel(x_sub, Mu, edge_index_sub, subset_idx, W, lin_l_w, lin_l_b,
           lin_r_w, lin_r_b, att):
    sidx_pad = jnp.zeros((WPAD,), jnp.int32).at[:N].set(
        subset_idx.astype(jnp.int32)).reshape(WPAD // 128, 128)
    w_sub = _wgather(sidx_pad, W)[:N]
    p_sub, l_log, x_l, x_r, cos, ll_raw = _dense_call(
        x_sub, Mu, w_sub, lin_l_w, lin_l_b.reshape(1, H),
        lin_r_w, lin_r_b.reshape(1, H))
    eidx_pad = jnp.zeros((2, EPAD), jnp.int32).at[:, :E].set(
        edge_index_sub.astype(jnp.int32)).reshape(2, EROWS, 128)
    vparts = _edges(eidx_pad, x_l, x_r, p_sub, att)[:, :N]
    ce_raw = _final_call(l_log, vparts)
    ll_prot = ll_raw[0, 0] / N
    ce_space = -ce_raw[0, 0] / N
    return (ll_prot, ce_space, p_sub)


# speedup vs baseline: 13.7162x; 1.4174x over previous
import functools

import jax
import jax.numpy as jnp
from jax import lax
from jax.experimental import pallas as pl
from jax.experimental.pallas import tpu as pltpu
from jax.experimental.pallas import tpu_sc as plsc

N = 10000
E = 320000
C = 32
G = 128
H = 8
NC, NS = 2, 16
NW = NC * NS

WROWS_PER_W = 8
WACT = 12
WPAD = WACT * WROWS_PER_W * 128
EROWS_PER_W = 80
EPAD = NW * EROWS_PER_W * 128
EROWS = EPAD // 128
CHUNK_ROWS = 8
NCHUNK = EROWS_PER_W // CHUNK_ROWS
NPAD = 10240
RPW = NPAD // NS

_mesh = plsc.VectorSubcoreMesh(core_axis_name="c", subcore_axis_name="s")


@functools.partial(
    pl.kernel,
    out_type=jax.ShapeDtypeStruct((WPAD, C), jnp.float32),
    mesh=_mesh,
    scratch_types=[
        pltpu.VMEM((WROWS_PER_W, 128), jnp.int32),
        pltpu.VMEM((WROWS_PER_W * 128, C), jnp.float32),
        pltpu.SemaphoreType.DMA,
    ],
    compiler_params=pltpu.CompilerParams(use_tc_tiling_on_sc=False, needs_layout_passes=False),
)
def _wgather(idx_hbm, w_hbm, out_hbm, idx_v, rows_v, sem):
    wid = lax.axis_index("s") * NC + lax.axis_index("c")

    @pl.when(wid < WACT)
    def _():
        pltpu.sync_copy(
            idx_hbm.at[pl.ds(wid * WROWS_PER_W, WROWS_PER_W), :], idx_v)
        cps = [
            pltpu.async_copy(w_hbm.at[idx_v.at[j]],
                             rows_v.at[pl.ds(j * 128, 128)], sem)
            for j in range(WROWS_PER_W)
        ]
        for cp in cps:
            cp.wait()
        pltpu.sync_copy(rows_v, out_hbm.at[pl.ds(wid * WROWS_PER_W * 128,
                                                 WROWS_PER_W * 128)])


def _dense_body(x_ref, mu_ref, ws_ref, llw_ref, llb_ref, lrw_ref, lrb_ref,
                p_ref, l_ref, xl_ref, xr_ref, cos_ref, ll_ref):
    i = pl.program_id(0)
    x = x_ref[...]
    xl_ref[...] = jnp.dot(x, llw_ref[...],
                          preferred_element_type=jnp.float32) + llb_ref[...]
    xr_ref[...] = jnp.dot(x, lrw_ref[...],
                          preferred_element_type=jnp.float32) + lrb_ref[...]
    w = ws_ref[...]
    ew = jnp.exp(w - jnp.max(w, axis=1, keepdims=True))
    p = ew / jnp.sum(ew, axis=1, keepdims=True)
    p_ref[...] = p
    l_ref[...] = jnp.log(p + 1e-8)
    xn = x / jnp.maximum(
        jnp.sqrt(jnp.sum(x * x, axis=1, keepdims=True)), 1e-12)
    mu = mu_ref[...]
    mun = mu / jnp.maximum(
        jnp.sqrt(jnp.sum(mu * mu, axis=1, keepdims=True)), 1e-12)
    cos = lax.dot_general(xn, mun, (((1,), (1,)), ((), ())),
                          preferred_element_type=jnp.float32)
    cos_ref[...] = cos

    @pl.when(i == 0)
    def _():
        ll_ref[0, 0] = 0.0

    ll_ref[0, 0] += jnp.sum(p * cos) * 10.0


_dense_call = pl.pallas_call(
    _dense_body,
    grid=(10,),
    in_specs=[
        pl.BlockSpec((1000, G), lambda i: (i, 0)),
        pl.BlockSpec((C, G), lambda i: (0, 0)),
        pl.BlockSpec((1000, C), lambda i: (i, 0)),
        pl.BlockSpec((G, H), lambda i: (0, 0)),
        pl.BlockSpec((1, H), lambda i: (0, 0)),
        pl.BlockSpec((G, H), lambda i: (0, 0)),
        pl.BlockSpec((1, H), lambda i: (0, 0)),
    ],
    out_specs=[
        pl.BlockSpec((1000, C), lambda i: (i, 0)),
        pl.BlockSpec((1000, C), lambda i: (i, 0)),
        pl.BlockSpec((1000, H), lambda i: (i, 0)),
        pl.BlockSpec((1000, H), lambda i: (i, 0)),
        pl.BlockSpec((1000, C), lambda i: (i, 0)),
        pl.BlockSpec((1, 1), lambda i: (0, 0), memory_space=pltpu.SMEM),
    ],
    out_shape=[
        jax.ShapeDtypeStruct((N, C), jnp.float32),
        jax.ShapeDtypeStruct((N, C), jnp.float32),
        jax.ShapeDtypeStruct((N, H), jnp.float32),
        jax.ShapeDtypeStruct((N, H), jnp.float32),
        jax.ShapeDtypeStruct((N, C), jnp.float32),
        jax.ShapeDtypeStruct((1, 1), jnp.float32),
    ],
)


@functools.partial(
    pl.kernel,
    out_type=jax.ShapeDtypeStruct((NC, NPAD, C), jnp.float32),
    mesh=_mesh,
    scratch_types=[
        pltpu.VMEM((2, CHUNK_ROWS, 128), jnp.int32),
        pltpu.VMEM((2, CHUNK_ROWS, 128), jnp.int32),
        pltpu.VMEM((2, CHUNK_ROWS * 128, H), jnp.float32),
        pltpu.VMEM((2, CHUNK_ROWS * 128, H), jnp.float32),
        pltpu.VMEM((CHUNK_ROWS * 128, C), jnp.float32),
        pltpu.VMEM((CHUNK_ROWS * 128,), jnp.float32),
        pltpu.VMEM((H,), jnp.float32),
        pltpu.VMEM_SHARED((NPAD, C), jnp.float32),
        pltpu.SemaphoreType.DMA,
        pltpu.SemaphoreType.DMA,
        pltpu.SemaphoreType.DMA,
    ],
    compiler_params=pltpu.CompilerParams(use_tc_tiling_on_sc=False, needs_layout_passes=False),
)
def _edges(eidx_hbm, xl_hbm, xr_hbm, p_hbm, att_hbm, vout_hbm,
           sidx, didx, xlr, xrr, prows, exb, attv, vsh, sem0, sem1, semp):
    cid = lax.axis_index("c")
    sid = lax.axis_index("s")
    wid = sid * NC + cid
    sems = (sem0, sem1)
    pltpu.sync_copy(att_hbm, attv)
    atts = [plsc.load_gather(attv, [jnp.full((16,), k, jnp.int32)])
            for k in range(H)]

    def zbody(r, carry):
        z = jnp.zeros((16,), jnp.float32)
        prows[r, pl.ds(0, 16)] = z
        prows[r, pl.ds(16, 16)] = z
        return carry

    lax.fori_loop(0, RPW, zbody, 0)
    pltpu.sync_copy(prows.at[pl.ds(0, RPW)], vsh.at[pl.ds(sid * RPW, RPW)])
    plsc.subcore_barrier()

    row0 = wid * EROWS_PER_W

    def issue(c, b):
        r0 = row0 + c * CHUNK_ROWS
        pltpu.sync_copy(eidx_hbm.at[0, pl.ds(r0, CHUNK_ROWS), :], sidx.at[b])
        pltpu.sync_copy(eidx_hbm.at[1, pl.ds(r0, CHUNK_ROWS), :], didx.at[b])
        cps = []
        for j in range(CHUNK_ROWS):
            cps.append(pltpu.async_copy(
                xl_hbm.at[sidx.at[b].at[j]],
                xlr.at[b].at[pl.ds(j * 128, 128)], sems[0]))
            cps.append(pltpu.async_copy(
                xr_hbm.at[didx.at[b].at[j]],
                xrr.at[b].at[pl.ds(j * 128, 128)], sems[0]))
        return cps

    issue(0, 0)

    def chunk_body(c, carry):
        b = c & 1
        r0 = row0 + c * CHUNK_ROWS
        for j in range(CHUNK_ROWS):
            pltpu.make_async_copy(
                xl_hbm.at[sidx.at[b].at[j]],
                xlr.at[b].at[pl.ds(j * 128, 128)], sems[0]).wait()
            pltpu.make_async_copy(
                xr_hbm.at[didx.at[b].at[j]],
                xrr.at[b].at[pl.ds(j * 128, 128)], sems[0]).wait()
        pcps = [pltpu.async_copy(
            p_hbm.at[sidx.at[b].at[j]],
            prows.at[pl.ds(j * 128, 128)], semp)
            for j in range(CHUNK_ROWS)]

        @pl.when(c + 1 < NCHUNK)
        def _():
            issue(c + 1, 1 - b)

        gbase = r0 * 128
        xlr_b, xrr_b = xlr.at[b], xrr.at[b]

        def ebody(g):
            ids = g * 16 + lax.iota(jnp.int32, 16)
            gid = gbase + ids
            mask = gid < E
            acc = jnp.zeros((16,), jnp.float32)
            for k in range(H):
                kf = jnp.full((16,), k, jnp.int32)
                a = plsc.load_gather(xlr_b, [ids, kf])
                bb = plsc.load_gather(xrr_b, [ids, kf])
                s = a + bb
                acc = acc + atts[k] * jnp.maximum(s, 0.2 * s)
            exb[pl.ds(g * 16, 16)] = jnp.where(mask, jnp.exp(acc), 0.0)

        plsc.parallel_loop(0, CHUNK_ROWS * 8, unroll=2)(ebody)
        for cp in pcps:
            cp.wait()

        def sbody(g):
            ids = g * 16 + lax.iota(jnp.int32, 16)
            ex = exb[pl.ds(g * 16, 16)]
            for k in range(C):
                kf = jnp.full((16,), k, jnp.int32)
                pk = plsc.load_gather(prows, [ids, kf])
                plsc.store_scatter(prows, [ids, kf], ex * pk)

        plsc.parallel_loop(0, CHUNK_ROWS * 8, unroll=2)(sbody)
        for j in range(CHUNK_ROWS):
            pltpu.sync_copy(prows.at[pl.ds(j * 128, 128)],
                            vsh.at[didx.at[b].at[j]], add=True)
        return carry

    lax.fori_loop(0, NCHUNK, chunk_body, 0)
    plsc.subcore_barrier()
    pltpu.sync_copy(vsh.at[pl.ds(sid * RPW, RPW)],
                    vout_hbm.at[cid, pl.ds(sid * RPW, RPW), :])


def _final_body(l_ref, v_ref, ce_ref):
    i = pl.program_id(0)
    v = v_ref[0] + v_ref[1]
    den = jnp.sum(v, axis=1, keepdims=True) + 1e-16
    part = jnp.sum(l_ref[...] * v / den)

    @pl.when(i == 0)
    def _():
        ce_ref[0, 0] = 0.0

    ce_ref[0, 0] += part


_final_call = pl.pallas_call(
    _final_body,
    grid=(10,),
    in_specs=[
        pl.BlockSpec((1000, C), lambda i: (i, 0)),
        pl.BlockSpec((NC, 1000, C), lambda i: (0, i, 0)),
    ],
    out_specs=pl.BlockSpec((1, 1), lambda i: (0, 0), memory_space=pltpu.SMEM),
    out_shape=jax.ShapeDtypeStruct((1, 1), jnp.float32),
)


def kernel(x_sub, Mu, edge_index_sub, subset_idx, W, lin_l_w, lin_l_b,
           lin_r_w, lin_r_b, att):
    sidx_pad = jnp.zeros((WPAD,), jnp.int32).at[:N].set(
        subset_idx.astype(jnp.int32)).reshape(WPAD // 128, 128)
    w_sub = _wgather(sidx_pad, W)[:N]
    p_sub, l_log, x_l, x_r, cos, ll_raw = _dense_call(
        x_sub, Mu, w_sub, lin_l_w, lin_l_b.reshape(1, H),
        lin_r_w, lin_r_b.reshape(1, H))
    eidx_pad = jnp.zeros((2, EPAD), jnp.int32).at[:, :E].set(
        edge_index_sub.astype(jnp.int32)).reshape(2, EROWS, 128)
    vparts = _edges(eidx_pad, x_l, x_r, p_sub, att)[:, :N]
    ce_raw = _final_call(l_log, vparts)
    ll_prot = ll_raw[0, 0] / N
    ce_space = -ce_raw[0, 0] / N
    return (ll_prot, ce_space, p_sub)


# speedup vs baseline: 15.5479x; 1.1335x over previous
import functools

import jax
import jax.numpy as jnp
from jax import lax
from jax.experimental import pallas as pl
from jax.experimental.pallas import tpu as pltpu
from jax.experimental.pallas import tpu_sc as plsc

N = 10000
E = 320000
C = 32
G = 128
H = 8
NC, NS = 2, 16
NW = NC * NS

WROWS_PER_W = 3
WACT = 32
WPAD = WACT * WROWS_PER_W * 128
EROWS_PER_W = 80
EPAD = NW * EROWS_PER_W * 128
EROWS = EPAD // 128
CHUNK_ROWS = 8
NCHUNK = EROWS_PER_W // CHUNK_ROWS
NPAD = 10240
RPW = NPAD // NS

_mesh = plsc.VectorSubcoreMesh(core_axis_name="c", subcore_axis_name="s")


@functools.partial(
    pl.kernel,
    out_type=jax.ShapeDtypeStruct((WPAD, C), jnp.float32),
    mesh=_mesh,
    scratch_types=[
        pltpu.VMEM((WROWS_PER_W, 128), jnp.int32),
        pltpu.VMEM((WROWS_PER_W * 128, C), jnp.float32),
        pltpu.SemaphoreType.DMA,
    ],
    compiler_params=pltpu.CompilerParams(use_tc_tiling_on_sc=False, needs_layout_passes=False),
)
def _wgather(idx_hbm, w_hbm, out_hbm, idx_v, rows_v, sem):
    wid = lax.axis_index("s") * NC + lax.axis_index("c")

    @pl.when(wid < WACT)
    def _():
        pltpu.sync_copy(
            idx_hbm.at[pl.ds(wid * WROWS_PER_W, WROWS_PER_W), :], idx_v)
        cps = [
            pltpu.async_copy(w_hbm.at[idx_v.at[j]],
                             rows_v.at[pl.ds(j * 128, 128)], sem)
            for j in range(WROWS_PER_W)
        ]
        for cp in cps:
            cp.wait()
        pltpu.sync_copy(rows_v, out_hbm.at[pl.ds(wid * WROWS_PER_W * 128,
                                                 WROWS_PER_W * 128)])


def _dense_body(x_ref, mu_ref, ws_ref, llw_ref, llb_ref, lrw_ref, lrb_ref,
                p_ref, l_ref, xl_ref, xr_ref, cos_ref, ll_ref):
    i = pl.program_id(0)
    x = x_ref[...]
    xl_ref[...] = jnp.dot(x, llw_ref[...],
                          preferred_element_type=jnp.float32) + llb_ref[...]
    xr_ref[...] = jnp.dot(x, lrw_ref[...],
                          preferred_element_type=jnp.float32) + lrb_ref[...]
    w = ws_ref[...]
    ew = jnp.exp(w - jnp.max(w, axis=1, keepdims=True))
    p = ew / jnp.sum(ew, axis=1, keepdims=True)
    p_ref[...] = p
    l_ref[...] = jnp.log(p + 1e-8)
    xn = x / jnp.maximum(
        jnp.sqrt(jnp.sum(x * x, axis=1, keepdims=True)), 1e-12)
    mu = mu_ref[...]
    mun = mu / jnp.maximum(
        jnp.sqrt(jnp.sum(mu * mu, axis=1, keepdims=True)), 1e-12)
    cos = lax.dot_general(xn, mun, (((1,), (1,)), ((), ())),
                          preferred_element_type=jnp.float32)
    cos_ref[...] = cos

    @pl.when(i == 0)
    def _():
        ll_ref[0, 0] = 0.0

    ll_ref[0, 0] += jnp.sum(p * cos) * 10.0


_dense_call = pl.pallas_call(
    _dense_body,
    grid=(10,),
    in_specs=[
        pl.BlockSpec((1000, G), lambda i: (i, 0)),
        pl.BlockSpec((C, G), lambda i: (0, 0)),
        pl.BlockSpec((1000, C), lambda i: (i, 0)),
        pl.BlockSpec((G, H), lambda i: (0, 0)),
        pl.BlockSpec((1, H), lambda i: (0, 0)),
        pl.BlockSpec((G, H), lambda i: (0, 0)),
        pl.BlockSpec((1, H), lambda i: (0, 0)),
    ],
    out_specs=[
        pl.BlockSpec((1000, C), lambda i: (i, 0)),
        pl.BlockSpec((1000, C), lambda i: (i, 0)),
        pl.BlockSpec((1000, H), lambda i: (i, 0)),
        pl.BlockSpec((1000, H), lambda i: (i, 0)),
        pl.BlockSpec((1000, C), lambda i: (i, 0)),
        pl.BlockSpec((1, 1), lambda i: (0, 0), memory_space=pltpu.SMEM),
    ],
    out_shape=[
        jax.ShapeDtypeStruct((N, C), jnp.float32),
        jax.ShapeDtypeStruct((N, C), jnp.float32),
        jax.ShapeDtypeStruct((N, H), jnp.float32),
        jax.ShapeDtypeStruct((N, H), jnp.float32),
        jax.ShapeDtypeStruct((N, C), jnp.float32),
        jax.ShapeDtypeStruct((1, 1), jnp.float32),
    ],
)


@functools.partial(
    pl.kernel,
    out_type=jax.ShapeDtypeStruct((NC, NPAD, C), jnp.float32),
    mesh=_mesh,
    scratch_types=[
        pltpu.VMEM((2, CHUNK_ROWS, 128), jnp.int32),
        pltpu.VMEM((2, CHUNK_ROWS, 128), jnp.int32),
        pltpu.VMEM((2, CHUNK_ROWS * 128, H), jnp.float32),
        pltpu.VMEM((2, CHUNK_ROWS * 128, H), jnp.float32),
        pltpu.VMEM((CHUNK_ROWS * 128, C), jnp.float32),
        pltpu.VMEM((CHUNK_ROWS * 128,), jnp.float32),
        pltpu.VMEM((H,), jnp.float32),
        pltpu.VMEM_SHARED((NPAD, C), jnp.float32),
        pltpu.SemaphoreType.DMA,
        pltpu.SemaphoreType.DMA,
        pltpu.SemaphoreType.DMA,
    ],
    compiler_params=pltpu.CompilerParams(use_tc_tiling_on_sc=False, needs_layout_passes=False),
)
def _edges(eidx_hbm, xl_hbm, xr_hbm, p_hbm, att_hbm, vout_hbm,
           sidx, didx, xlr, xrr, prows, exb, attv, vsh, sem0, sem1, semp):
    cid = lax.axis_index("c")
    sid = lax.axis_index("s")
    wid = sid * NC + cid
    sems = (sem0, sem1)
    pltpu.sync_copy(att_hbm, attv)
    atts = [plsc.load_gather(attv, [jnp.full((16,), k, jnp.int32)])
            for k in range(H)]

    def zbody(r, carry):
        z = jnp.zeros((16,), jnp.float32)
        prows[r, pl.ds(0, 16)] = z
        prows[r, pl.ds(16, 16)] = z
        return carry

    lax.fori_loop(0, RPW, zbody, 0)
    pltpu.sync_copy(prows.at[pl.ds(0, RPW)], vsh.at[pl.ds(sid * RPW, RPW)])
    plsc.subcore_barrier()

    row0 = wid * EROWS_PER_W

    def issue(c, b):
        r0 = row0 + c * CHUNK_ROWS
        pltpu.sync_copy(eidx_hbm.at[0, pl.ds(r0, CHUNK_ROWS), :], sidx.at[b])
        pltpu.sync_copy(eidx_hbm.at[1, pl.ds(r0, CHUNK_ROWS), :], didx.at[b])
        cps = []
        for j in range(CHUNK_ROWS):
            cps.append(pltpu.async_copy(
                xl_hbm.at[sidx.at[b].at[j]],
                xlr.at[b].at[pl.ds(j * 128, 128)], sems[0]))
            cps.append(pltpu.async_copy(
                xr_hbm.at[didx.at[b].at[j]],
                xrr.at[b].at[pl.ds(j * 128, 128)], sems[0]))
        return cps

    issue(0, 0)

    def chunk_body(c, carry):
        b = c & 1
        r0 = row0 + c * CHUNK_ROWS
        for j in range(CHUNK_ROWS):
            pltpu.make_async_copy(
                xl_hbm.at[sidx.at[b].at[j]],
                xlr.at[b].at[pl.ds(j * 128, 128)], sems[0]).wait()
            pltpu.make_async_copy(
                xr_hbm.at[didx.at[b].at[j]],
                xrr.at[b].at[pl.ds(j * 128, 128)], sems[0]).wait()
        pcps = [pltpu.async_copy(
            p_hbm.at[sidx.at[b].at[j]],
            prows.at[pl.ds(j * 128, 128)], semp)
            for j in range(CHUNK_ROWS)]

        @pl.when(c + 1 < NCHUNK)
        def _():
            issue(c + 1, 1 - b)

        gbase = r0 * 128
        xlr_b, xrr_b = xlr.at[b], xrr.at[b]

        def ebody(g):
            ids = g * 16 + lax.iota(jnp.int32, 16)
            gid = gbase + ids
            mask = gid < E
            acc = jnp.zeros((16,), jnp.float32)
            for k in range(H):
                kf = jnp.full((16,), k, jnp.int32)
                a = plsc.load_gather(xlr_b, [ids, kf])
                bb = plsc.load_gather(xrr_b, [ids, kf])
                s = a + bb
                acc = acc + atts[k] * jnp.maximum(s, 0.2 * s)
            exb[pl.ds(g * 16, 16)] = jnp.where(mask, jnp.exp(acc), 0.0)

        plsc.parallel_loop(0, CHUNK_ROWS * 8, unroll=4)(ebody)
        for cp in pcps:
            cp.wait()

        def sbody(g):
            ids = g * 16 + lax.iota(jnp.int32, 16)
            ex = exb[pl.ds(g * 16, 16)]
            for k in range(C):
                kf = jnp.full((16,), k, jnp.int32)
                pk = plsc.load_gather(prows, [ids, kf])
                plsc.store_scatter(prows, [ids, kf], ex * pk)

        plsc.parallel_loop(0, CHUNK_ROWS * 8, unroll=4)(sbody)
        for j in range(CHUNK_ROWS):
            pltpu.sync_copy(prows.at[pl.ds(j * 128, 128)],
                            vsh.at[didx.at[b].at[j]], add=True)
        return carry

    lax.fori_loop(0, NCHUNK, chunk_body, 0)
    plsc.subcore_barrier()
    pltpu.sync_copy(vsh.at[pl.ds(sid * RPW, RPW)],
                    vout_hbm.at[cid, pl.ds(sid * RPW, RPW), :])


def _final_body(l_ref, v_ref, ce_ref):
    i = pl.program_id(0)
    v = v_ref[0] + v_ref[1]
    den = jnp.sum(v, axis=1, keepdims=True) + 1e-16
    part = jnp.sum(l_ref[...] * v / den)

    @pl.when(i == 0)
    def _():
        ce_ref[0, 0] = 0.0

    ce_ref[0, 0] += part


_final_call = pl.pallas_call(
    _final_body,
    grid=(10,),
    in_specs=[
        pl.BlockSpec((1000, C), lambda i: (i, 0)),
        pl.BlockSpec((NC, 1000, C), lambda i: (0, i, 0)),
    ],
    out_specs=pl.BlockSpec((1, 1), lambda i: (0, 0), memory_space=pltpu.SMEM),
    out_shape=jax.ShapeDtypeStruct((1, 1), jnp.float32),
)


def kernel(x_sub, Mu, edge_index_sub, subset_idx, W, lin_l_w, lin_l_b,
           lin_r_w, lin_r_b, att):
    sidx_pad = jnp.zeros((WPAD,), jnp.int32).at[:N].set(
        subset_idx.astype(jnp.int32)).reshape(WPAD // 128, 128)
    w_sub = _wgather(sidx_pad, W)[:N]
    p_sub, l_log, x_l, x_r, cos, ll_raw = _dense_call(
        x_sub, Mu, w_sub, lin_l_w, lin_l_b.reshape(1, H),
        lin_r_w, lin_r_b.reshape(1, H))
    eidx_pad = jnp.zeros((2, EPAD), jnp.int32).at[:, :E].set(
        edge_index_sub.astype(jnp.int32)).reshape(2, EROWS, 128)
    vparts = _edges(eidx_pad, x_l, x_r, p_sub, att)
    ce_raw = _final_call(l_log, vparts)
    ll_prot = ll_raw[0, 0] / N
    ce_space = -ce_raw[0, 0] / N
    return (ll_prot, ce_space, p_sub)


# speedup vs baseline: 23.9749x; 1.5420x over previous
import functools

import jax
import jax.numpy as jnp
from jax import lax
from jax.experimental import pallas as pl
from jax.experimental.pallas import tpu as pltpu
from jax.experimental.pallas import tpu_sc as plsc

N = 10000
E = 320000
C = 32
G = 128
H = 8
NC, NS = 2, 16
NW = NC * NS

WROWS_PER_W = 3
WACT = 32
WPAD = WACT * WROWS_PER_W * 128
EROWS_PER_W = 79
EROWS = 2536
EPAD = EROWS * 128
CHUNK_ROWS = 8
NCHUNK = 10
RPW = N // NS
CP = 33
HP = 9

_mesh = plsc.VectorSubcoreMesh(core_axis_name="c", subcore_axis_name="s")


@functools.partial(
    pl.kernel,
    out_type=jax.ShapeDtypeStruct((WPAD, C), jnp.float32),
    mesh=_mesh,
    scratch_types=[
        pltpu.VMEM((WROWS_PER_W, 128), jnp.int32),
        pltpu.VMEM((WROWS_PER_W * 128, C), jnp.float32),
        pltpu.SemaphoreType.DMA,
    ],
    compiler_params=pltpu.CompilerParams(use_tc_tiling_on_sc=False, needs_layout_passes=False),
)
def _wgather(idx_hbm, w_hbm, out_hbm, idx_v, rows_v, sem):
    wid = lax.axis_index("s") * NC + lax.axis_index("c")

    @pl.when(wid < WACT)
    def _():
        pltpu.sync_copy(
            idx_hbm.at[pl.ds(wid * WROWS_PER_W, WROWS_PER_W), :], idx_v)
        cps = [
            pltpu.async_copy(w_hbm.at[idx_v.at[j]],
                             rows_v.at[pl.ds(j * 128, 128)], sem)
            for j in range(WROWS_PER_W)
        ]
        for cp in cps:
            cp.wait()
        pltpu.sync_copy(rows_v, out_hbm.at[pl.ds(wid * WROWS_PER_W * 128,
                                                 WROWS_PER_W * 128)])


def _dense_body(x_ref, mu_ref, ws_ref, llw_ref, llb_ref, lrw_ref, lrb_ref,
                p_ref, l_ref, xl_ref, xr_ref, cos_ref, ll_ref):
    i = pl.program_id(0)
    x = x_ref[...]
    z1 = jnp.zeros((x.shape[0], 1), jnp.float32)
    xl_ref[...] = jnp.concatenate(
        [jnp.dot(x, llw_ref[...],
                 preferred_element_type=jnp.float32) + llb_ref[...], z1], axis=1)
    xr_ref[...] = jnp.concatenate(
        [jnp.dot(x, lrw_ref[...],
                 preferred_element_type=jnp.float32) + lrb_ref[...], z1], axis=1)
    w = ws_ref[...]
    ew = jnp.exp(w - jnp.max(w, axis=1, keepdims=True))
    p = ew / jnp.sum(ew, axis=1, keepdims=True)
    p_ref[...] = p
    l_ref[...] = jnp.log(p + 1e-8)
    xn = x / jnp.maximum(
        jnp.sqrt(jnp.sum(x * x, axis=1, keepdims=True)), 1e-12)
    mu = mu_ref[...]
    mun = mu / jnp.maximum(
        jnp.sqrt(jnp.sum(mu * mu, axis=1, keepdims=True)), 1e-12)
    cos = lax.dot_general(xn, mun, (((1,), (1,)), ((), ())),
                          preferred_element_type=jnp.float32)
    cos_ref[...] = cos

    @pl.when(i == 0)
    def _():
        ll_ref[0, 0] = 0.0

    ll_ref[0, 0] += jnp.sum(p * cos) * 10.0


_dense_call = pl.pallas_call(
    _dense_body,
    grid=(10,),
    in_specs=[
        pl.BlockSpec((1000, G), lambda i: (i, 0)),
        pl.BlockSpec((C, G), lambda i: (0, 0)),
        pl.BlockSpec((1000, C), lambda i: (i, 0)),
        pl.BlockSpec((G, H), lambda i: (0, 0)),
        pl.BlockSpec((1, H), lambda i: (0, 0)),
        pl.BlockSpec((G, H), lambda i: (0, 0)),
        pl.BlockSpec((1, H), lambda i: (0, 0)),
    ],
    out_specs=[
        pl.BlockSpec((1000, C), lambda i: (i, 0)),
        pl.BlockSpec((1000, C), lambda i: (i, 0)),
        pl.BlockSpec((1000, HP), lambda i: (i, 0)),
        pl.BlockSpec((1000, HP), lambda i: (i, 0)),
        pl.BlockSpec((1000, C), lambda i: (i, 0)),
        pl.BlockSpec((1, 1), lambda i: (0, 0), memory_space=pltpu.SMEM),
    ],
    out_shape=[
        jax.ShapeDtypeStruct((N, C), jnp.float32),
        jax.ShapeDtypeStruct((N, C), jnp.float32),
        jax.ShapeDtypeStruct((N, HP), jnp.float32),
        jax.ShapeDtypeStruct((N, HP), jnp.float32),
        jax.ShapeDtypeStruct((N, C), jnp.float32),
        jax.ShapeDtypeStruct((1, 1), jnp.float32),
    ],
)


@functools.partial(
    pl.kernel,
    out_type=jax.ShapeDtypeStruct((NC, N, C), jnp.float32),
    mesh=_mesh,
    scratch_types=[
        pltpu.VMEM((2, CHUNK_ROWS, 128), jnp.int32),
        pltpu.VMEM((2, CHUNK_ROWS, 128), jnp.int32),
        pltpu.VMEM((2, CHUNK_ROWS * 128, HP), jnp.float32),
        pltpu.VMEM((2, CHUNK_ROWS * 128, HP), jnp.float32),
        pltpu.VMEM((CHUNK_ROWS * 128, C), jnp.float32),
        pltpu.VMEM((CHUNK_ROWS * 128,), jnp.float32),
        pltpu.VMEM((H,), jnp.float32),
        pltpu.VMEM_SHARED((N, C), jnp.float32),
        pltpu.SemaphoreType.DMA,
        pltpu.SemaphoreType.DMA,
        pltpu.SemaphoreType.DMA,
    ],
    compiler_params=pltpu.CompilerParams(use_tc_tiling_on_sc=False, needs_layout_passes=False),
)
def _edges(eidx_hbm, xl_hbm, xr_hbm, p_hbm, att_hbm, vout_hbm,
           sidx, didx, xlr, xrr, prows, exb, attv, vsh, sem0, sem1, semp):
    cid = lax.axis_index("c")
    sid = lax.axis_index("s")
    wid = sid * NC + cid
    sems = (sem0, sem1)
    pltpu.sync_copy(att_hbm, attv)
    atts = [plsc.load_gather(attv, [jnp.full((16,), k, jnp.int32)])
            for k in range(H)]

    def zbody(r, carry):
        z = jnp.zeros((16,), jnp.float32)
        prows[r, pl.ds(0, 16)] = z
        prows[r, pl.ds(16, 16)] = z
        return carry

    lax.fori_loop(0, RPW, zbody, 0)
    pltpu.sync_copy(prows.at[pl.ds(0, RPW)], vsh.at[pl.ds(sid * RPW, RPW)])
    plsc.subcore_barrier()

    row0 = wid * EROWS_PER_W
    wend = jnp.minimum((row0 + EROWS_PER_W) * 128, E)

    def issue(c, b):
        r0 = row0 + c * CHUNK_ROWS
        pltpu.sync_copy(eidx_hbm.at[0, pl.ds(r0, CHUNK_ROWS), :], sidx.at[b])
        pltpu.sync_copy(eidx_hbm.at[1, pl.ds(r0, CHUNK_ROWS), :], didx.at[b])
        cps = []
        for j in range(CHUNK_ROWS):
            cps.append(pltpu.async_copy(
                xl_hbm.at[sidx.at[b].at[j]],
                xlr.at[b].at[pl.ds(j * 128, 128)], sems[0]))
            cps.append(pltpu.async_copy(
                xr_hbm.at[didx.at[b].at[j]],
                xrr.at[b].at[pl.ds(j * 128, 128)], sems[0]))
        return cps

    issue(0, 0)

    def chunk_body(c, carry):
        b = c & 1
        r0 = row0 + c * CHUNK_ROWS
        for j in range(CHUNK_ROWS):
            pltpu.make_async_copy(
                xl_hbm.at[sidx.at[b].at[j]],
                xlr.at[b].at[pl.ds(j * 128, 128)], sems[0]).wait()
            pltpu.make_async_copy(
                xr_hbm.at[didx.at[b].at[j]],
                xrr.at[b].at[pl.ds(j * 128, 128)], sems[0]).wait()
        pcps = [pltpu.async_copy(
            p_hbm.at[sidx.at[b].at[j]],
            prows.at[pl.ds(j * 128, 128)], semp)
            for j in range(CHUNK_ROWS)]

        @pl.when(c + 1 < NCHUNK)
        def _():
            issue(c + 1, 1 - b)

        gbase = r0 * 128
        xlr_b, xrr_b = xlr.at[b], xrr.at[b]

        def ebody(g):
            ids = g * 16 + lax.iota(jnp.int32, 16)
            gid = gbase + ids
            mask = gid < wend
            acc = jnp.zeros((16,), jnp.float32)
            for k in range(H):
                kf = jnp.full((16,), k, jnp.int32)
                a = plsc.load_gather(xlr_b, [ids, kf])
                bb = plsc.load_gather(xrr_b, [ids, kf])
                s = a + bb
                acc = acc + atts[k] * jnp.maximum(s, 0.2 * s)
            exb[pl.ds(g * 16, 16)] = jnp.where(mask, jnp.exp(acc), 0.0)

        plsc.parallel_loop(0, CHUNK_ROWS * 8, unroll=4)(ebody)
        for cp in pcps:
            cp.wait()

        def sbody(i):
            exs = plsc.load_gather(exb, [jnp.full((16,), 0, jnp.int32) + i])
            a = prows[i, pl.ds(0, 16)]
            bq = prows[i, pl.ds(16, 16)]
            prows[i, pl.ds(0, 16)] = exs * a
            prows[i, pl.ds(16, 16)] = exs * bq

        plsc.parallel_loop(0, CHUNK_ROWS * 128, unroll=4)(sbody)
        for j in range(CHUNK_ROWS):
            pltpu.sync_copy(prows.at[pl.ds(j * 128, 128)],
                            vsh.at[didx.at[b].at[j]], add=True)
        return carry

    lax.fori_loop(0, NCHUNK, chunk_body, 0)
    plsc.subcore_barrier()
    pltpu.sync_copy(vsh.at[pl.ds(sid * RPW, RPW)],
                    vout_hbm.at[cid, pl.ds(sid * RPW, RPW), :])


def _final_body(l_ref, v_ref, ce_ref):
    i = pl.program_id(0)
    v = v_ref[0] + v_ref[1]
    den = jnp.sum(v, axis=1, keepdims=True) + 1e-16
    part = jnp.sum(l_ref[...] * v / den)

    @pl.when(i == 0)
    def _():
        ce_ref[0, 0] = 0.0

    ce_ref[0, 0] += part


_final_call = pl.pallas_call(
    _final_body,
    grid=(10,),
    in_specs=[
        pl.BlockSpec((1000, C), lambda i: (i, 0)),
        pl.BlockSpec((NC, 1000, C), lambda i: (0, i, 0)),
    ],
    out_specs=pl.BlockSpec((1, 1), lambda i: (0, 0), memory_space=pltpu.SMEM),
    out_shape=jax.ShapeDtypeStruct((1, 1), jnp.float32),
)


def kernel(x_sub, Mu, edge_index_sub, subset_idx, W, lin_l_w, lin_l_b,
           lin_r_w, lin_r_b, att):
    sidx_pad = jnp.zeros((WPAD,), jnp.int32).at[:N].set(
        subset_idx.astype(jnp.int32)).reshape(WPAD // 128, 128)
    w_sub = _wgather(sidx_pad, W)[:N]
    p_sub, l_log, x_l, x_r, cos, ll_raw = _dense_call(
        x_sub, Mu, w_sub, lin_l_w, lin_l_b.reshape(1, H),
        lin_r_w, lin_r_b.reshape(1, H))
    eidx_pad = jnp.zeros((2, EPAD), jnp.int32).at[:, :E].set(
        edge_index_sub.astype(jnp.int32)).reshape(2, EROWS, 128)
    vparts = _edges(eidx_pad, x_l, x_r, p_sub, att)
    ce_raw = _final_call(l_log, vparts)
    ll_prot = ll_raw[0, 0] / N
    ce_space = -ce_raw[0, 0] / N
    return (ll_prot, ce_space, p_sub)
